# baseline (speedup 1.0000x reference)
import functools

import jax
import jax.numpy as jnp
from jax import lax
from jax.experimental import pallas as pl
from jax.experimental.pallas import tpu as pltpu
from jax.experimental.pallas import tpu_sc as plsc

VOCAB = 1000000
EMBED = 32
B, F, P = 4096, 26, 20
BF = B * F
NW = 32
WPW = BF // NW
N = 64
C = WPW // N
IDS = N * P
IDS_PAD = IDS + 32
G = 128
NG = IDS // G
HALF = EMBED // 2


def _body(ids_hbm, table_hbm, out_hbm,
          idx0, idx1, rows0, rows1, out_v, inv_v, sem0, sem1):
    wid = lax.axis_index("s") * 2 + lax.axis_index("c")
    row0 = wid * WPW

    def issue(c, idx_v, rows_v, sem):
        base = (row0 + c * N) * P
        pltpu.sync_copy(ids_hbm.at[pl.ds(base, IDS)], idx_v.at[pl.ds(0, IDS)])
        for g in range(NG):
            pltpu.async_copy(
                table_hbm.at[idx_v.at[pl.ds(g * G, G)]],
                rows_v.at[pl.ds(g * G, G)], sem)

    def drain(idx_v, rows_v, sem):
        for g in range(NG):
            pltpu.make_async_copy(
                table_hbm.at[idx_v.at[pl.ds(g * G, G)]],
                rows_v.at[pl.ds(g * G, G)], sem).wait()

    lane = lax.iota(jnp.int32, 16)

    def compute(c, idx_v, rows_v):
        def blk_body(jj, carry2):
            n0 = jj * 16
            base = n0 * P
            zmin = idx_v[pl.ds(base, 16)]
            for q in range(1, P):
                g = idx_v[pl.ds(base + q * 16, 16)]
                zmin = jnp.minimum(zmin, g)
            hz = zmin[0]
            for q in range(1, 16):
                hz = jnp.minimum(hz, zmin[q])
            has_zero = hz == 0

            inv_v[...] = jnp.full((16,), 1.0 / P, jnp.float32)

            @pl.when(has_zero)
            def _():
                cnt16 = jnp.zeros((16,), jnp.float32)
                for l in range(16):
                    off = (n0 + l) * P
                    ind1 = jnp.where(idx_v[pl.ds(off, 16)] != 0, 1, 0)
                    ind2 = jnp.where(idx_v[pl.ds(off + 16, 16)] != 0, 1, 0)
                    cnt_i = ind1[0]
                    for q in range(1, 16):
                        cnt_i = cnt_i + ind1[q]
                    for q in range(P - 16):
                        cnt_i = cnt_i + ind2[q]
                    cnt16 = jnp.where(lane == l, cnt_i.astype(jnp.float32),
                                      cnt16)
                inv_v[...] = 1.0 / jnp.maximum(cnt16, 1.0)

            inv16 = inv_v[...]
            for l in range(16):
                inv_s = inv16[l]
                r = (n0 + l) * P
                for h in range(2):
                    acc = rows_v[r, pl.ds(h * HALF, HALF)]
                    for p in range(1, P):
                        acc = acc + rows_v[r + p, pl.ds(h * HALF, HALF)]
                    out_v[n0 + l, pl.ds(h * HALF, HALF)] = acc * inv_s
            return carry2

        lax.fori_loop(0, N // 16, blk_body, 0)
        pltpu.sync_copy(out_v, out_hbm.at[pl.ds(row0 + c * N, N)])

    issue(0, idx0, rows0, sem0)

    def pair_body(i, carry):
        c0 = i * 2
        issue(c0 + 1, idx1, rows1, sem1)
        drain(idx0, rows0, sem0)
        compute(c0, idx0, rows0)

        @pl.when(c0 + 2 < C)
        def _():
            issue(c0 + 2, idx0, rows0, sem0)

        drain(idx1, rows1, sem1)
        compute(c0 + 1, idx1, rows1)
        return carry

    lax.fori_loop(0, C // 2, pair_body, 0)


def kernel(path_ids, table):
    ids_flat = path_ids.reshape(-1).astype(jnp.int32)
    mesh = plsc.VectorSubcoreMesh(core_axis_name="c", subcore_axis_name="s")
    run = functools.partial(
        pl.kernel,
        out_type=jax.ShapeDtypeStruct((BF, EMBED), jnp.float32),
        mesh=mesh,
        compiler_params=pltpu.CompilerParams(use_tc_tiling_on_sc=False),
        scratch_types=[
            pltpu.VMEM((IDS_PAD,), jnp.int32),
            pltpu.VMEM((IDS_PAD,), jnp.int32),
            pltpu.VMEM((IDS, EMBED), jnp.float32),
            pltpu.VMEM((IDS, EMBED), jnp.float32),
            pltpu.VMEM((N, EMBED), jnp.float32),
            pltpu.VMEM((16,), jnp.float32),
            pltpu.SemaphoreType.DMA,
            pltpu.SemaphoreType.DMA,
        ],
    )(_body)
    out = run(ids_flat, table)
    return out.reshape(B, F, EMBED)


# speedup vs baseline: 1.3163x; 1.3163x over previous
import functools

import jax
import jax.numpy as jnp
from jax import lax
from jax.experimental import pallas as pl
from jax.experimental.pallas import tpu as pltpu
from jax.experimental.pallas import tpu_sc as plsc

VOCAB = 1000000
EMBED = 32
B, F, P = 4096, 26, 20
BF = B * F
NW = 32
WPW = BF // NW
N = 64
C = WPW // N
IDS = N * P
G = 128
NG = IDS // G
HALF = EMBED // 2
PADC = 128
RSUB = PADC // EMBED


def _body(ids_hbm, table_hbm, out_hbm,
          idx0, idx1, rows0, rows1, out_v, sem0, sem1):
    wid = lax.axis_index("s") * 2 + lax.axis_index("c")
    row0 = wid * WPW

    def issue(c, idx_v, rows_v, sem):
        base = (row0 + c * N) * P
        pltpu.sync_copy(ids_hbm.at[pl.ds(base, IDS)], idx_v)
        for g in range(NG):
            pltpu.async_copy(
                table_hbm.at[idx_v.at[pl.ds(g * G, G)]],
                rows_v.at[pl.ds(g * G, G)], sem)

    def drain(idx_v, rows_v, sem):
        for g in range(NG):
            pltpu.make_async_copy(
                table_hbm.at[idx_v.at[pl.ds(g * G, G)]],
                rows_v.at[pl.ds(g * G, G)], sem).wait()

    def compute(c, idx_v, rows_v):
        def blk_body(jj, carry2):
            n0 = jj * 16
            cnt = jnp.zeros((16,), jnp.float32)
            for p in range(P):
                g = idx_v[pl.ds(p * N + n0, 16)]
                cnt = cnt + jnp.where(g != 0, 1.0, 0.0)
            inv16 = 1.0 / jnp.maximum(cnt, 1.0)
            for l in range(16):
                inv_s = inv16[l]
                for h in range(2):
                    acc = rows_v[n0 + l, pl.ds(h * HALF, HALF)]
                    for p in range(1, P):
                        acc = acc + rows_v[p * N + n0 + l, pl.ds(h * HALF, HALF)]
                    out_v[n0 + l, pl.ds(h * HALF, HALF)] = acc * inv_s
            return carry2

        lax.fori_loop(0, N // 16, blk_body, 0)
        pltpu.sync_copy(out_v, out_hbm.at[pl.ds(row0 + c * N, N)])

    issue(0, idx0, rows0, sem0)

    def pair_body(i, carry):
        c0 = i * 2
        issue(c0 + 1, idx1, rows1, sem1)
        drain(idx0, rows0, sem0)
        compute(c0, idx0, rows0)

        @pl.when(c0 + 2 < C)
        def _():
            issue(c0 + 2, idx0, rows0, sem0)

        drain(idx1, rows1, sem1)
        compute(c0 + 1, idx1, rows1)
        return carry

    lax.fori_loop(0, C // 2, pair_body, 0)


def kernel(path_ids, table):
    ids_flat = (path_ids.reshape(NW, C, N, P)
                .transpose(0, 1, 3, 2)
                .reshape(-1)
                .astype(jnp.int32)) * RSUB
    tab4 = jnp.pad(table, ((0, 0), (0, PADC - EMBED))).reshape(
        VOCAB * RSUB, EMBED)
    mesh = plsc.VectorSubcoreMesh(core_axis_name="c", subcore_axis_name="s")
    run = functools.partial(
        pl.kernel,
        out_type=jax.ShapeDtypeStruct((BF, EMBED), jnp.float32),
        mesh=mesh,
        compiler_params=pltpu.CompilerParams(use_tc_tiling_on_sc=False),
        scratch_types=[
            pltpu.VMEM((IDS,), jnp.int32),
            pltpu.VMEM((IDS,), jnp.int32),
            pltpu.VMEM((IDS, EMBED), jnp.float32),
            pltpu.VMEM((IDS, EMBED), jnp.float32),
            pltpu.VMEM((N, EMBED), jnp.float32),
            pltpu.SemaphoreType.DMA,
            pltpu.SemaphoreType.DMA,
        ],
    )(_body)
    out = run(ids_flat, tab4)
    return out.reshape(B, F, EMBED)


# speedup vs baseline: 1.3967x; 1.0611x over previous
import functools

import jax
import jax.numpy as jnp
from jax import lax
from jax.experimental import pallas as pl
from jax.experimental.pallas import tpu as pltpu
from jax.experimental.pallas import tpu_sc as plsc

VOCAB = 1000000
EMBED = 32
B, F, P = 4096, 26, 20
BF = B * F
NW = 32
N = 64
NB = B // N
NCHUNK = F * NB
CPW = NCHUNK // NW
IDS = N * P
G = 128
NG = IDS // G
HALF = EMBED // 2
PADC = 128
RSUB = PADC // EMBED


def _body(ids_hbm, table_hbm, out_hbm,
          idx0, idx1, sidx0, sidx1, rows0, rows1, out_v, sem0, sem1):
    wid = lax.axis_index("s") * 2 + lax.axis_index("c")
    q0 = wid * CPW

    def issue(q, idx_v, sidx_v, rows_v, sem):
        f = q // NB
        b0 = (q % NB) * N
        pltpu.sync_copy(ids_hbm.at[pl.ds(f * P, P), pl.ds(b0, N)], idx_v)
        for p in range(P):
            for j in range(N // 16):
                g = idx_v[p, pl.ds(j * 16, 16)]
                sidx_v[pl.ds(p * N + j * 16, 16)] = g * RSUB
        for g in range(NG):
            pltpu.async_copy(
                table_hbm.at[sidx_v.at[pl.ds(g * G, G)]],
                rows_v.at[pl.ds(g * G, G)], sem)

    def drain(sidx_v, rows_v, sem):
        for g in range(NG):
            pltpu.make_async_copy(
                table_hbm.at[sidx_v.at[pl.ds(g * G, G)]],
                rows_v.at[pl.ds(g * G, G)], sem).wait()

    def compute(q, idx_v, rows_v):
        def blk_body(jj, carry2):
            n0 = jj * 16
            cnt = jnp.zeros((16,), jnp.float32)
            for p in range(P):
                g = idx_v[p, pl.ds(n0, 16)]
                cnt = cnt + jnp.where(g != 0, 1.0, 0.0)
            inv16 = 1.0 / jnp.maximum(cnt, 1.0)
            for l in range(16):
                inv_s = inv16[l]
                for h in range(2):
                    acc = rows_v[n0 + l, pl.ds(h * HALF, HALF)]
                    for p in range(1, P):
                        acc = acc + rows_v[p * N + n0 + l, pl.ds(h * HALF, HALF)]
                    out_v[n0 + l, pl.ds(h * HALF, HALF)] = acc * inv_s
            return carry2

        lax.fori_loop(0, N // 16, blk_body, 0)
        f = q // NB
        b0 = (q % NB) * N
        pltpu.sync_copy(out_v, out_hbm.at[pl.ds(b0, N), f])

    issue(q0, idx0, sidx0, rows0, sem0)

    def pair_body(i, carry):
        c0 = q0 + i * 2
        issue(c0 + 1, idx1, sidx1, rows1, sem1)
        drain(sidx0, rows0, sem0)
        compute(c0, idx0, rows0)

        @pl.when(i * 2 + 2 < CPW)
        def _():
            issue(c0 + 2, idx0, sidx0, rows0, sem0)

        drain(sidx1, rows1, sem1)
        compute(c0 + 1, idx1, rows1)
        return carry

    lax.fori_loop(0, CPW // 2, pair_body, 0)


def kernel(path_ids, table):
    ids_fp_b = jnp.transpose(path_ids, (1, 2, 0)).reshape(F * P, B)
    tab4 = jnp.pad(table, ((0, 0), (0, PADC - EMBED))).reshape(
        VOCAB * RSUB, EMBED)
    mesh = plsc.VectorSubcoreMesh(core_axis_name="c", subcore_axis_name="s")
    run = functools.partial(
        pl.kernel,
        out_type=jax.ShapeDtypeStruct((B, F, EMBED), jnp.float32),
        mesh=mesh,
        compiler_params=pltpu.CompilerParams(use_tc_tiling_on_sc=False),
        scratch_types=[
            pltpu.VMEM((P, N), jnp.int32),
            pltpu.VMEM((P, N), jnp.int32),
            pltpu.VMEM((IDS,), jnp.int32),
            pltpu.VMEM((IDS,), jnp.int32),
            pltpu.VMEM((IDS, EMBED), jnp.float32),
            pltpu.VMEM((IDS, EMBED), jnp.float32),
            pltpu.VMEM((N, EMBED), jnp.float32),
            pltpu.SemaphoreType.DMA,
            pltpu.SemaphoreType.DMA,
        ],
    )(_body)
    return run(ids_fp_b, tab4)


# speedup vs baseline: 1.6698x; 1.1956x over previous
import functools

import jax
import jax.numpy as jnp
from jax import lax
from jax.experimental import pallas as pl
from jax.experimental.pallas import tpu as pltpu
from jax.experimental.pallas import tpu_sc as plsc

VOCAB = 1000000
EMBED = 32
B, F, P = 4096, 26, 20
BF = B * F
NW = 32
N = 64
NB = B // N
NCHUNK = F * NB
CPW = NCHUNK // NW
IDS = N * P
HALF = EMBED // 2
PADC = 128
RSUB = PADC // EMBED


def _body(ids_hbm, table_hbm, out_hbm,
          idx0, idx1, sidx0, sidx1, acc0, acc1, sem0, sem1):
    wid = lax.axis_index("s") * 2 + lax.axis_index("c")
    q0 = wid * CPW
    zero16 = jnp.zeros((16,), jnp.float32)

    def issue(q, idx_v, sidx_v, acc_v, sem):
        f = q // NB
        b0 = (q % NB) * N
        pltpu.sync_copy(ids_hbm.at[pl.ds(f * P, P), pl.ds(b0, N)], idx_v)

        def zero_body(n, carry):
            acc_v[n, pl.ds(0, 16)] = zero16
            acc_v[n, pl.ds(16, 16)] = zero16
            return carry

        lax.fori_loop(0, N, zero_body, 0)
        for p in range(P):
            for j in range(N // 16):
                g = idx_v[p, pl.ds(j * 16, 16)]
                sidx_v[pl.ds(p * N + j * 16, 16)] = g * RSUB
        for p in range(P):
            pltpu.async_copy(
                table_hbm.at[sidx_v.at[pl.ds(p * N, N)]],
                acc_v, sem, add=True)

    def drain(sidx_v, acc_v, sem):
        for p in range(P):
            pltpu.make_async_copy(
                table_hbm.at[sidx_v.at[pl.ds(p * N, N)]],
                acc_v, sem).wait()

    def compute(q, idx_v, acc_v):
        def blk_body(jj, carry2):
            n0 = jj * 16
            cnt = jnp.zeros((16,), jnp.float32)
            for p in range(P):
                g = idx_v[p, pl.ds(n0, 16)]
                cnt = cnt + jnp.where(g != 0, 1.0, 0.0)
            inv16 = 1.0 / jnp.maximum(cnt, 1.0)
            for l in range(16):
                inv_s = inv16[l]
                for h in range(2):
                    acc_v[n0 + l, pl.ds(h * HALF, HALF)] = (
                        acc_v[n0 + l, pl.ds(h * HALF, HALF)] * inv_s)
            return carry2

        lax.fori_loop(0, N // 16, blk_body, 0)
        f = q // NB
        b0 = (q % NB) * N
        pltpu.sync_copy(acc_v, out_hbm.at[pl.ds(b0, N), f])

    issue(q0, idx0, sidx0, acc0, sem0)

    def pair_body(i, carry):
        c0 = q0 + i * 2
        issue(c0 + 1, idx1, sidx1, acc1, sem1)
        drain(sidx0, acc0, sem0)
        compute(c0, idx0, acc0)

        @pl.when(i * 2 + 2 < CPW)
        def _():
            issue(c0 + 2, idx0, sidx0, acc0, sem0)

        drain(sidx1, acc1, sem1)
        compute(c0 + 1, idx1, acc1)
        return carry

    lax.fori_loop(0, CPW // 2, pair_body, 0)


def kernel(path_ids, table):
    ids_fp_b = jnp.transpose(path_ids, (1, 2, 0)).reshape(F * P, B)
    tab4 = jnp.pad(table, ((0, 0), (0, PADC - EMBED))).reshape(
        VOCAB * RSUB, EMBED)
    mesh = plsc.VectorSubcoreMesh(core_axis_name="c", subcore_axis_name="s")
    run = functools.partial(
        pl.kernel,
        out_type=jax.ShapeDtypeStruct((B, F, EMBED), jnp.float32),
        mesh=mesh,
        compiler_params=pltpu.CompilerParams(use_tc_tiling_on_sc=False),
        scratch_types=[
            pltpu.VMEM((P, N), jnp.int32),
            pltpu.VMEM((P, N), jnp.int32),
            pltpu.VMEM((IDS,), jnp.int32),
            pltpu.VMEM((IDS,), jnp.int32),
            pltpu.VMEM((N, EMBED), jnp.float32),
            pltpu.VMEM((N, EMBED), jnp.float32),
            pltpu.SemaphoreType.DMA,
            pltpu.SemaphoreType.DMA,
        ],
    )(_body)
    return run(ids_fp_b, tab4)
